# baseline (speedup 1.0000x reference)
import functools

import jax
import jax.numpy as jnp
from jax import lax
from jax.experimental import pallas as pl
from jax.experimental.pallas import tpu as pltpu
from jax.experimental.pallas import tpu_sc as plsc

DIM = 64
NSEQ = 4096
SEQ = 200
B = NSEQ * SEQ
VOC = 1000000
VALIGN = (VOC // 128) * 128
NBLK = VALIGN // 128
PAIRS = VALIGN // 2
NC, NS = 2, 16
NW = NC * NS
SEQ_PER_W = NSEQ // NW
TOK_PER_W = SEQ_PER_W * SEQ
LANES = 16
NGRP = SEQ_PER_W // LANES
NBLKW = NBLK // NW + 1

_mesh = plsc.VectorSubcoreMesh(core_axis_name="c", subcore_axis_name="s")
_BLK_CUT = NBLK - NW * (NBLK // NW)


@functools.partial(
    pl.kernel,
    mesh=_mesh,
    out_type=jax.ShapeDtypeStruct((PAIRS, 128), jnp.float32),
    scratch_types=[
        pltpu.VMEM((2, 64, 128), jnp.float32),
        pltpu.VMEM((64, 128), jnp.float32),
        pltpu.SemaphoreType.DMA((2,)),
    ],
    compiler_params=pltpu.CompilerParams(needs_layout_passes=False),
)
def _fmt(wt_hbm, out_hbm, in_v, slab_v, sem_in):
    wid = lax.axis_index("s") * NC + lax.axis_index("c")
    nblk = jnp.where(wid < _BLK_CUT, NBLKW, NBLKW - 1)
    bstart = jnp.where(
        wid < _BLK_CUT,
        wid * NBLKW,
        _BLK_CUT * NBLKW + (wid - _BLK_CUT) * (NBLKW - 1),
    )
    last = bstart + nblk - 1

    def blk_at(i):
        return jnp.minimum(bstart + i, last)

    def in_copy(blk, b):
        c0 = pl.multiple_of(blk * 128, 128)
        return pltpu.make_async_copy(
            wt_hbm.at[pl.ds(0, DIM), pl.ds(c0, 128)], in_v.at[b], sem_in.at[b],
        )

    lane = lax.iota(jnp.int32, LANES)
    row_s = lane >> 1
    colb_s = (lane & 1) * DIM

    in_copy(blk_at(0), 0).start()
    in_copy(blk_at(1), 1).start()

    def pair_step(i2, carry):
        for b in range(2):
            i = i2 * 2 + b
            blk = blk_at(i)
            in_copy(blk, b).wait()

            def transpose_d8(d8, c):
                for dd in range(8):
                    d = d8 * 8 + dd
                    for g in range(8):
                        val = in_v[b, d, pl.ds(g * LANES, LANES)]
                        plsc.store_scatter(
                            slab_v, [g * 8 + row_s, colb_s + d], val,
                        )
                return c

            lax.fori_loop(0, DIM // 8, transpose_d8, 0)

            p0 = pl.multiple_of(blk * 64, 64)
            pltpu.sync_copy(slab_v, out_hbm.at[pl.ds(p0, 64)])
            in_copy(blk_at(i + 2), b).start()
        return carry

    lax.fori_loop(0, (NBLKW + 1) // 2, pair_step, 0)
    in_copy(last, 0).wait()
    in_copy(last, 1).wait()


@functools.partial(
    pl.kernel,
    mesh=_mesh,
    out_type=jax.ShapeDtypeStruct((SEQ, DIM, NSEQ), jnp.float32),
    scratch_types=[
        pltpu.VMEM((TOK_PER_W,), jnp.int32),
        pltpu.VMEM((64, 64), jnp.float32),
        pltpu.VMEM((2, 192, 128), jnp.float32),
        pltpu.VMEM((2, 128), jnp.int32),
        pltpu.VMEM((64, 128), jnp.float32),
        pltpu.SemaphoreType.DMA((2,)),
    ],
    compiler_params=pltpu.CompilerParams(needs_layout_passes=False),
)
def _emb(idx_hbm, tbl_hbm, side_hbm, out_hbm,
         tok_v, side_v, gbuf_v, pidx_v, slab_v, sem_g):
    wid = lax.axis_index("s") * NC + lax.axis_index("c")
    s0 = pl.multiple_of(wid * SEQ_PER_W, SEQ_PER_W)

    pltpu.sync_copy(idx_hbm.at[pl.ds(wid * TOK_PER_W, TOK_PER_W)], tok_v)
    pltpu.sync_copy(side_hbm, side_v)

    lane = lax.iota(jnp.int32, LANES)
    lane200 = lane * SEQ

    def park(r, c):
        for b in range(2):
            for g in range(4):
                v = side_v[r, pl.ds(g * LANES, LANES)]
                gbuf_v[b, 128 + r, pl.ds(g * LANES, LANES)] = v
        return c

    lax.fori_loop(0, 64, park, 0)

    def build_idx(t, b):
        for g in range(NGRP):
            tv = plsc.load_gather(tok_v, [g * LANES * SEQ + lane200 + t])
            pidx_v[b, pl.ds(g * LANES, LANES)] = jnp.minimum(
                tv >> 1, PAIRS - 1
            )

    def gather(b):
        return pltpu.make_async_copy(
            tbl_hbm.at[pidx_v.at[b]], gbuf_v.at[b, pl.ds(0, 128)], sem_g.at[b],
        )

    build_idx(0, 0)
    gather(0).start()
    build_idx(1, 1)
    gather(1).start()

    def pair_step(t2, carry):
        for b in range(2):
            t = t2 * 2 + b
            gather(b).wait()

            for g in range(NGRP):
                tv = plsc.load_gather(tok_v, [g * LANES * SEQ + lane200 + t])
                tail = tv >= VALIGN
                row = jnp.where(tail, 128 + (tv - VALIGN), g * LANES + lane)
                colb = jnp.where(tail, 0, (tv & 1) * DIM)

                def per_d8(d8, c):
                    for dd in range(8):
                        d = d8 * 8 + dd
                        val = plsc.load_gather(gbuf_v.at[b], [row, colb + d])
                        slab_v[d, pl.ds(g * LANES, LANES)] = val
                    return c

                lax.fori_loop(0, DIM // 8, per_d8, 0)

            pltpu.sync_copy(slab_v, out_hbm.at[t, :, pl.ds(s0, SEQ_PER_W)])

            tn = jnp.minimum(t + 2, SEQ - 1)
            build_idx(tn, b)
            gather(b).start()
        return carry

    lax.fori_loop(0, SEQ // 2, pair_step, 0)
    gather(0).wait()
    gather(1).wait()


def kernel(token_ids, weight):
    idx = token_ids.reshape(B).astype(jnp.int32)
    wt = weight.T
    tbl = _fmt(wt)
    side = weight[VALIGN:, :]
    out_t = _emb(idx, tbl, side)
    return jnp.transpose(out_t, (2, 0, 1))


# speedup vs baseline: 2.1454x; 2.1454x over previous
import functools

import jax
import jax.numpy as jnp
from jax import lax
from jax.experimental import pallas as pl
from jax.experimental.pallas import tpu as pltpu
from jax.experimental.pallas import tpu_sc as plsc

DIM = 64
NSEQ = 4096
SEQ = 200
B = NSEQ * SEQ
NC, NS = 2, 16
NW = NC * NS
SEQ_PER_W = NSEQ // NW
CH = SEQ
NCH = SEQ_PER_W
SUBS = (128, 72)
NBUF = 4

_mesh = plsc.VectorSubcoreMesh(core_axis_name="c", subcore_axis_name="s")


@functools.partial(
    pl.kernel,
    mesh=_mesh,
    out_type=jax.ShapeDtypeStruct((NSEQ, SEQ, DIM), jnp.float32),
    scratch_types=[
        pltpu.VMEM((NBUF, CH), jnp.int32),
        pltpu.VMEM((NBUF, CH, DIM), jnp.float32),
        pltpu.SemaphoreType.DMA((NBUF,)),
        pltpu.SemaphoreType.DMA,
        pltpu.SemaphoreType.DMA((NBUF,)),
    ],
    compiler_params=pltpu.CompilerParams(use_tc_tiling_on_sc=False),
)
def _emb(idx_hbm, table_hbm, out_hbm, idx_v, rows_v, sem_idx, sem_g, sem_out):
    wid = lax.axis_index("s") * NC + lax.axis_index("c")
    seq_base = wid * SEQ_PER_W
    base = seq_base * SEQ

    def idx_copy(g, b):
        return pltpu.make_async_copy(
            idx_hbm.at[pl.ds(base + g * CH, CH)], idx_v.at[b], sem_idx.at[b],
        )

    def out_copy(g, b):
        return pltpu.make_async_copy(
            rows_v.at[b], out_hbm.at[seq_base + g], sem_out.at[b],
        )

    def gather_copies(b):
        off = 0
        copies = []
        for n in SUBS:
            copies.append(pltpu.make_async_copy(
                table_hbm.at[idx_v.at[b, pl.ds(off, n)]],
                rows_v.at[b, pl.ds(off, n)],
                sem_g,
            ))
            off += n
        return copies

    def fire_gathers(b):
        for cp in gather_copies(b):
            cp.start()

    def drain_gathers(b):
        for cp in gather_copies(b):
            cp.wait()


    for b in range(NBUF):
        idx_copy(b, b).start()
    idx_copy(0, 0).wait()
    fire_gathers(0)

    def slot_head(g):
        idx_copy(g + 1, (g + 1) % NBUF).wait()
        fire_gathers((g + 1) % NBUF)
        drain_gathers(g % NBUF)
        idx_copy(g + NBUF, g % NBUF).start()
        out_copy(g, g % NBUF).start()

    def slot_steady(g):
        idx_copy(g + 1, (g + 1) % NBUF).wait()
        out_copy(g + 1 - NBUF, (g + 1) % NBUF).wait()
        fire_gathers((g + 1) % NBUF)
        drain_gathers(g % NBUF)
        idx_copy(g + NBUF, g % NBUF).start()
        out_copy(g, g % NBUF).start()

    def slot_tail(g, last):
        if not last:
            idx_copy(g + 1, (g + 1) % NBUF).wait()
            out_copy(g + 1 - NBUF, (g + 1) % NBUF).wait()
            fire_gathers((g + 1) % NBUF)
        drain_gathers(g % NBUF)
        out_copy(g, g % NBUF).start()

    for g in range(NBUF):
        if g < NBUF - 1:
            slot_head(g)
        else:
            slot_steady(g)

    def step(i, carry):
        g0 = NBUF + i * NBUF
        for b in range(NBUF):
            slot_steady(g0 + b)
        return carry

    lax.fori_loop(0, (NCH - 2 * NBUF) // NBUF, step, 0)

    for g in range(NCH - NBUF, NCH):
        slot_tail(g, last=(g == NCH - 1))

    for g in range(NCH - NBUF, NCH):
        out_copy(g, g % NBUF).wait()


def kernel(token_ids, weight):
    idx = token_ids.reshape(B).astype(jnp.int32)
    return _emb(idx, weight)
